# baseline (speedup 1.0000x reference)
import functools

import jax
import jax.numpy as jnp
from jax import lax
from jax.experimental import pallas as pl
from jax.experimental.pallas import tpu as pltpu
from jax.experimental.pallas import tpu_sc as plsc

N = 10000
E = 320000
D = 128
H = 128
G = 64
OUT = 16

NC = 2
NS = 16
NW = NC * NS
CH = 128
NCH = 80
E_PAD = NW * NCH * CH
N2 = N + 16
N_PAD = 10240
ROWS_PT = N_PAD // NS
ZR = 128


def _sc_segsum_body(h_hbm, src_hbm, dst_hbm, out,
                    sidx, didx, rows0, rows1, acc, sem0, sem1):
    cid = lax.axis_index("c")
    sid = lax.axis_index("s")
    wid = sid * NC + cid

    def zrow(i, carry):
        for c in range(H // 16):
            rows0[i, pl.ds(c * 16, 16)] = jnp.zeros((16,), jnp.float32)
        return carry
    lax.fori_loop(0, ZR, zrow, 0)
    for j in range(ROWS_PT // ZR):
        pltpu.sync_copy(rows0, acc.at[pl.ds(sid * ROWS_PT + j * ZR, ZR)])
    plsc.subcore_barrier()

    HN = NCH // 2
    for half in range(2):
        pltpu.sync_copy(src_hbm.at[wid, pl.ds(half * HN, HN)], sidx)
        pltpu.sync_copy(dst_hbm.at[wid, pl.ds(half * HN, HN)], didx)

        pltpu.async_copy(h_hbm.at[sidx.at[0]], rows0, sem0)

        def body(b, carry):
            i0 = 2 * b
            pltpu.async_copy(h_hbm.at[sidx.at[i0 + 1]], rows1, sem1)
            pltpu.make_async_copy(h_hbm.at[sidx.at[i0]], rows0, sem0).wait()

            @pl.when(i0 + 2 < HN)
            def _():
                pltpu.async_copy(h_hbm.at[sidx.at[i0 + 2]], rows0, sem0)
            pltpu.make_async_copy(h_hbm.at[sidx.at[i0 + 1]], rows1,
                                  sem1).wait()
            return carry
        lax.fori_loop(0, HN // 2, body, 0)
    plsc.subcore_barrier()

    sl = pl.ds(sid * ROWS_PT, ROWS_PT)
    pltpu.sync_copy(acc.at[sl], out.at[cid, sl])


@functools.lru_cache(maxsize=None)
def _build_sc_segsum():
    mesh = plsc.VectorSubcoreMesh(core_axis_name="c", subcore_axis_name="s",
                                  num_cores=NC, num_subcores=NS)
    return pl.kernel(
        _sc_segsum_body,
        out_type=jax.ShapeDtypeStruct((2, N_PAD, H), jnp.float32),
        mesh=mesh,
        scratch_types=[
            pltpu.VMEM((NCH // 2, CH), jnp.int32),
            pltpu.VMEM((NCH // 2, CH), jnp.int32),
            pltpu.VMEM((CH, H), jnp.float32),
            pltpu.VMEM((CH, H), jnp.float32),
            pltpu.VMEM_SHARED((N_PAD, H), jnp.float32),
            pltpu.SemaphoreType.DMA,
            pltpu.SemaphoreType.DMA,
        ],
    )


def _sc_segsum(h, src3, dst3):
    return _build_sc_segsum()(h, src3, dst3)


def _mlp_body(h_ref, a_ref, w1_ref, b1_ref, g_ref, bt_ref,
              w2_ref, b2_ref, o_ref):
    hin = (h_ref[pl.ds(0, N), :] + a_ref[0, pl.ds(0, N), :]
           + a_ref[1, pl.ds(0, N), :])
    h1 = jnp.dot(hin, w1_ref[...], preferred_element_type=jnp.float32,
                 precision=lax.Precision.HIGHEST) + b1_ref[...]
    mu = jnp.mean(h1, axis=0, keepdims=True)
    var = jnp.mean(jnp.square(h1 - mu), axis=0, keepdims=True)
    hn = (h1 - mu) * lax.rsqrt(var + 1e-5) * g_ref[...] + bt_ref[...]
    hn = jnp.maximum(hn, 0.0)
    h2 = jnp.dot(hn, w2_ref[...], preferred_element_type=jnp.float32,
                 precision=lax.Precision.HIGHEST) + b2_ref[...]
    o_ref[pl.ds(0, N), :] = jnp.maximum(h2, 0.0)
    o_ref[pl.ds(N, N2 - N), :] = jnp.zeros((N2 - N, H), jnp.float32)


_mlp = pl.pallas_call(
    _mlp_body,
    out_shape=jax.ShapeDtypeStruct((N2, H), jnp.float32),
)


def _pool_readout_body(h1_ref, h2_ref, h3_ref, b_ref, w1_ref, b1_ref,
                       w2_ref, b2_ref, o1_ref, o2_ref):
    gids = lax.broadcasted_iota(jnp.int32, (N, G), 1)
    onehot = (b_ref[...] == gids).astype(jnp.float32)
    dn = (((0,), (0,)), ((), ()))
    ones = jnp.ones((N, 1), jnp.float32)
    cnt = lax.dot_general(onehot, ones, dn,
                          preferred_element_type=jnp.float32,
                          precision=lax.Precision.HIGHEST)
    cnt = jnp.maximum(cnt, 1.0)
    s1 = lax.dot_general(onehot, h1_ref[pl.ds(0, N), :], dn,
                         preferred_element_type=jnp.float32,
                         precision=lax.Precision.HIGHEST)
    s2 = lax.dot_general(onehot, h2_ref[pl.ds(0, N), :], dn,
                         preferred_element_type=jnp.float32,
                         precision=lax.Precision.HIGHEST)
    s3 = lax.dot_general(onehot, h3_ref[pl.ds(0, N), :], dn,
                         preferred_element_type=jnp.float32,
                         precision=lax.Precision.HIGHEST)
    hcat = jnp.concatenate([s1, s2, s3], axis=1) / cnt
    hl = jnp.dot(hcat, w1_ref[...], preferred_element_type=jnp.float32,
                 precision=lax.Precision.HIGHEST) + b1_ref[...]
    hl = jnp.maximum(hl, 0.0)
    ho = jnp.dot(hl, w2_ref[...], preferred_element_type=jnp.float32,
                 precision=lax.Precision.HIGHEST) + b2_ref[...]
    o1_ref[...] = ho
    mx = jnp.max(ho, axis=1, keepdims=True)
    lse = jnp.log(jnp.sum(jnp.exp(ho - mx), axis=1, keepdims=True)) + mx
    o2_ref[...] = ho - lse


_pool_readout = pl.pallas_call(
    _pool_readout_body,
    out_shape=(jax.ShapeDtypeStruct((G, OUT), jnp.float32),
               jax.ShapeDtypeStruct((G, OUT), jnp.float32)),
)


def kernel(x, L0_W1, L0_b1, L0_g, L0_bt, L0_W2, L0_b2,
           L1_W1, L1_b1, L1_g, L1_bt, L1_W2, L1_b2,
           L2_W1, L2_b1, L2_g, L2_bt, L2_W2, L2_b2,
           lin1_W, lin1_b, lin2_W, lin2_b, edge_index, batch):
    pad = jnp.full((E_PAD - E,), N, jnp.int32)
    src3 = jnp.concatenate([edge_index[0], pad]).reshape(NW, NCH, CH)
    dst3 = jnp.concatenate([edge_index[1], pad]).reshape(NW, NCH, CH)
    params = [
        (L0_W1, L0_b1, L0_g, L0_bt, L0_W2, L0_b2),
        (L1_W1, L1_b1, L1_g, L1_bt, L1_W2, L1_b2),
        (L2_W1, L2_b1, L2_g, L2_bt, L2_W2, L2_b2),
    ]
    h = jnp.concatenate([x, jnp.zeros((N2 - N, D), jnp.float32)])
    hs = []
    for (W1, b1, g, bt, W2, b2) in params:
        agg = _sc_segsum(h, src3, dst3)
        h = _mlp(h, agg, W1, b1.reshape(1, H), g.reshape(1, H),
                 bt.reshape(1, H), W2, b2.reshape(1, H))
        hs.append(h)
    return _pool_readout(hs[0], hs[1], hs[2], batch.reshape(N, 1),
                         lin1_W, lin1_b.reshape(1, H * 3),
                         lin2_W, lin2_b.reshape(1, OUT))


# speedup vs baseline: 1.0023x; 1.0023x over previous
import functools

import jax
import jax.numpy as jnp
from jax import lax
from jax.experimental import pallas as pl
from jax.experimental.pallas import tpu as pltpu
from jax.experimental.pallas import tpu_sc as plsc

N = 10000
E = 320000
D = 128
H = 128
G = 64
OUT = 16

NC = 2
NS = 16
NW = NC * NS
CH = 128
NCH = 80
E_PAD = NW * NCH * CH
N2 = N + 16
N_PAD = 10240
ROWS_PT = N_PAD // NS
ZR = 128


def _sc_segsum_body(h_hbm, src_hbm, dst_hbm, out,
                    sidx, didx, sflat0, sflat1, rows0, rows1, acc,
                    sem0, sem1):
    cid = lax.axis_index("c")
    sid = lax.axis_index("s")
    wid = sid * NC + cid

    def zrow(i, carry):
        for c in range(H // 16):
            rows0[i, pl.ds(c * 16, 16)] = jnp.zeros((16,), jnp.float32)
        return carry
    lax.fori_loop(0, ZR, zrow, 0)
    for j in range(ROWS_PT // ZR):
        pltpu.sync_copy(rows0, acc.at[pl.ds(sid * ROWS_PT + j * ZR, ZR)])
    plsc.subcore_barrier()

    HN = NCH // 2
    for half in range(2):
        pltpu.sync_copy(src_hbm.at[wid, pl.ds(half * HN, HN)], sidx)
        pltpu.sync_copy(dst_hbm.at[wid, pl.ds(half * HN, HN)], didx)

        def stage(dst_flat, i):
            for c in range(CH // 16):
                dst_flat[pl.ds(c * 16, 16)] = sidx[i, pl.ds(c * 16, 16)]

        stage(sflat0, 0)
        pltpu.async_copy(h_hbm.at[sflat0], rows0, sem0)

        def body(b, carry):
            i0 = 2 * b
            stage(sflat1, i0 + 1)
            pltpu.async_copy(h_hbm.at[sflat1], rows1, sem1)
            pltpu.make_async_copy(h_hbm.at[sflat0], rows0, sem0).wait()
            pltpu.sync_copy(rows0, acc.at[didx.at[i0]], add=True)

            @pl.when(i0 + 2 < HN)
            def _():
                stage(sflat0, i0 + 2)
                pltpu.async_copy(h_hbm.at[sflat0], rows0, sem0)
            pltpu.make_async_copy(h_hbm.at[sflat1], rows1, sem1).wait()
            pltpu.sync_copy(rows1, acc.at[didx.at[i0 + 1]], add=True)
            return carry
        lax.fori_loop(0, HN // 2, body, 0)
    plsc.subcore_barrier()

    sl = pl.ds(sid * ROWS_PT, ROWS_PT)
    pltpu.sync_copy(acc.at[sl], out.at[cid, sl])


@functools.lru_cache(maxsize=None)
def _build_sc_segsum():
    mesh = plsc.VectorSubcoreMesh(core_axis_name="c", subcore_axis_name="s",
                                  num_cores=NC, num_subcores=NS)
    return pl.kernel(
        _sc_segsum_body,
        out_type=jax.ShapeDtypeStruct((2, N_PAD, H), jnp.float32),
        mesh=mesh,
        scratch_types=[
            pltpu.VMEM((NCH // 2, CH), jnp.int32),
            pltpu.VMEM((NCH // 2, CH), jnp.int32),
            pltpu.VMEM((CH,), jnp.int32),
            pltpu.VMEM((CH,), jnp.int32),
            pltpu.VMEM((CH, H), jnp.float32),
            pltpu.VMEM((CH, H), jnp.float32),
            pltpu.VMEM_SHARED((N_PAD, H), jnp.float32),
            pltpu.SemaphoreType.DMA,
            pltpu.SemaphoreType.DMA,
        ],
    )


def _sc_segsum(h, src3, dst3):
    return _build_sc_segsum()(h, src3, dst3)


def _mlp_body(h_ref, a_ref, w1_ref, b1_ref, g_ref, bt_ref,
              w2_ref, b2_ref, o_ref):
    hin = (h_ref[pl.ds(0, N), :] + a_ref[0, pl.ds(0, N), :]
           + a_ref[1, pl.ds(0, N), :])
    h1 = jnp.dot(hin, w1_ref[...], preferred_element_type=jnp.float32,
                 precision=lax.Precision.HIGHEST) + b1_ref[...]
    mu = jnp.mean(h1, axis=0, keepdims=True)
    var = jnp.mean(jnp.square(h1 - mu), axis=0, keepdims=True)
    hn = (h1 - mu) * lax.rsqrt(var + 1e-5) * g_ref[...] + bt_ref[...]
    hn = jnp.maximum(hn, 0.0)
    h2 = jnp.dot(hn, w2_ref[...], preferred_element_type=jnp.float32,
                 precision=lax.Precision.HIGHEST) + b2_ref[...]
    o_ref[pl.ds(0, N), :] = jnp.maximum(h2, 0.0)
    o_ref[pl.ds(N, N2 - N), :] = jnp.zeros((N2 - N, H), jnp.float32)


_mlp = pl.pallas_call(
    _mlp_body,
    out_shape=jax.ShapeDtypeStruct((N2, H), jnp.float32),
)


def _pool_readout_body(h1_ref, h2_ref, h3_ref, b_ref, w1_ref, b1_ref,
                       w2_ref, b2_ref, o1_ref, o2_ref):
    gids = lax.broadcasted_iota(jnp.int32, (N, G), 1)
    onehot = (b_ref[...] == gids).astype(jnp.float32)
    dn = (((0,), (0,)), ((), ()))
    ones = jnp.ones((N, 1), jnp.float32)
    cnt = lax.dot_general(onehot, ones, dn,
                          preferred_element_type=jnp.float32,
                          precision=lax.Precision.HIGHEST)
    cnt = jnp.maximum(cnt, 1.0)
    s1 = lax.dot_general(onehot, h1_ref[pl.ds(0, N), :], dn,
                         preferred_element_type=jnp.float32,
                         precision=lax.Precision.HIGHEST)
    s2 = lax.dot_general(onehot, h2_ref[pl.ds(0, N), :], dn,
                         preferred_element_type=jnp.float32,
                         precision=lax.Precision.HIGHEST)
    s3 = lax.dot_general(onehot, h3_ref[pl.ds(0, N), :], dn,
                         preferred_element_type=jnp.float32,
                         precision=lax.Precision.HIGHEST)
    hcat = jnp.concatenate([s1, s2, s3], axis=1) / cnt
    hl = jnp.dot(hcat, w1_ref[...], preferred_element_type=jnp.float32,
                 precision=lax.Precision.HIGHEST) + b1_ref[...]
    hl = jnp.maximum(hl, 0.0)
    ho = jnp.dot(hl, w2_ref[...], preferred_element_type=jnp.float32,
                 precision=lax.Precision.HIGHEST) + b2_ref[...]
    o1_ref[...] = ho
    mx = jnp.max(ho, axis=1, keepdims=True)
    lse = jnp.log(jnp.sum(jnp.exp(ho - mx), axis=1, keepdims=True)) + mx
    o2_ref[...] = ho - lse


_pool_readout = pl.pallas_call(
    _pool_readout_body,
    out_shape=(jax.ShapeDtypeStruct((G, OUT), jnp.float32),
               jax.ShapeDtypeStruct((G, OUT), jnp.float32)),
)


def kernel(x, L0_W1, L0_b1, L0_g, L0_bt, L0_W2, L0_b2,
           L1_W1, L1_b1, L1_g, L1_bt, L1_W2, L1_b2,
           L2_W1, L2_b1, L2_g, L2_bt, L2_W2, L2_b2,
           lin1_W, lin1_b, lin2_W, lin2_b, edge_index, batch):
    pad = jnp.full((E_PAD - E,), N, jnp.int32)
    src3 = jnp.concatenate([edge_index[0], pad]).reshape(NW, NCH, CH)
    dst3 = jnp.concatenate([edge_index[1], pad]).reshape(NW, NCH, CH)
    params = [
        (L0_W1, L0_b1, L0_g, L0_bt, L0_W2, L0_b2),
        (L1_W1, L1_b1, L1_g, L1_bt, L1_W2, L1_b2),
        (L2_W1, L2_b1, L2_g, L2_bt, L2_W2, L2_b2),
    ]
    h = jnp.concatenate([x, jnp.zeros((N2 - N, D), jnp.float32)])
    hs = []
    for (W1, b1, g, bt, W2, b2) in params:
        agg = _sc_segsum(h, src3, dst3)
        h = _mlp(h, agg, W1, b1.reshape(1, H), g.reshape(1, H),
                 bt.reshape(1, H), W2, b2.reshape(1, H))
        hs.append(h)
    return _pool_readout(hs[0], hs[1], hs[2], batch.reshape(N, 1),
                         lin1_W, lin1_b.reshape(1, H * 3),
                         lin2_W, lin2_b.reshape(1, OUT))


# speedup vs baseline: 1.0707x; 1.0682x over previous
import functools

import jax
import jax.numpy as jnp
from jax import lax
from jax.experimental import pallas as pl
from jax.experimental.pallas import tpu as pltpu
from jax.experimental.pallas import tpu_sc as plsc

N = 10000
E = 320000
D = 128
H = 128
G = 64
OUT = 16

NC = 2
NS = 16
NW = NC * NS
CH = 80
NCH = 128
E_PAD = NW * NCH * CH
N2 = N + 16
N_PAD = 10240
ROWS_PT = N_PAD // NS
ZR = CH


def _sc_segsum_body(h_hbm, src_hbm, dst_hbm, out,
                    sidx, didx, sflat0, sflat1, rows0, rows1, acc,
                    sem0, sem1):
    cid = lax.axis_index("c")
    sid = lax.axis_index("s")
    wid = sid * NC + cid

    def zrow(i, carry):
        for c in range(H // 16):
            rows0[i, pl.ds(c * 16, 16)] = jnp.zeros((16,), jnp.float32)
        return carry
    lax.fori_loop(0, ZR, zrow, 0)
    for j in range(ROWS_PT // ZR):
        pltpu.sync_copy(rows0, acc.at[pl.ds(sid * ROWS_PT + j * ZR, ZR)])
    plsc.subcore_barrier()

    HN = NCH // 2
    for half in range(2):
        pltpu.sync_copy(src_hbm.at[wid, pl.ds(half * HN, HN)], sidx)
        pltpu.sync_copy(dst_hbm.at[wid, pl.ds(half * HN, HN)], didx)

        def stage(dst_flat, i):
            for c in range(CH // 16):
                dst_flat[pl.ds(c * 16, 16)] = sidx[i, pl.ds(c * 16, 16)]

        stage(sflat0, 0)
        pltpu.async_copy(h_hbm.at[sflat0], rows0, sem0)

        def body(b, carry):
            i0 = 2 * b
            stage(sflat1, i0 + 1)
            pltpu.async_copy(h_hbm.at[sflat1], rows1, sem1)
            pltpu.make_async_copy(h_hbm.at[sflat0], rows0, sem0).wait()
            pltpu.sync_copy(rows0, acc.at[didx.at[i0]], add=True)

            @pl.when(i0 + 2 < HN)
            def _():
                stage(sflat0, i0 + 2)
                pltpu.async_copy(h_hbm.at[sflat0], rows0, sem0)
            pltpu.make_async_copy(h_hbm.at[sflat1], rows1, sem1).wait()
            pltpu.sync_copy(rows1, acc.at[didx.at[i0 + 1]], add=True)
            return carry
        lax.fori_loop(0, HN // 2, body, 0)
    plsc.subcore_barrier()

    sl = pl.ds(sid * ROWS_PT, ROWS_PT)
    pltpu.sync_copy(acc.at[sl], out.at[cid, sl])


@functools.lru_cache(maxsize=None)
def _build_sc_segsum():
    mesh = plsc.VectorSubcoreMesh(core_axis_name="c", subcore_axis_name="s",
                                  num_cores=NC, num_subcores=NS)
    return pl.kernel(
        _sc_segsum_body,
        out_type=jax.ShapeDtypeStruct((2, N_PAD, H), jnp.float32),
        mesh=mesh,
        scratch_types=[
            pltpu.VMEM((NCH // 2, CH), jnp.int32),
            pltpu.VMEM((NCH // 2, CH), jnp.int32),
            pltpu.VMEM((CH,), jnp.int32),
            pltpu.VMEM((CH,), jnp.int32),
            pltpu.VMEM((CH, H), jnp.float32),
            pltpu.VMEM((CH, H), jnp.float32),
            pltpu.VMEM_SHARED((N_PAD, H), jnp.float32),
            pltpu.SemaphoreType.DMA,
            pltpu.SemaphoreType.DMA,
        ],
    )


def _sc_segsum(h, src3, dst3):
    return _build_sc_segsum()(h, src3, dst3)


def _mlp_body(h_ref, a_ref, w1_ref, b1_ref, g_ref, bt_ref,
              w2_ref, b2_ref, o_ref):
    hin = (h_ref[pl.ds(0, N), :] + a_ref[0, pl.ds(0, N), :]
           + a_ref[1, pl.ds(0, N), :])
    h1 = jnp.dot(hin, w1_ref[...], preferred_element_type=jnp.float32,
                 precision=lax.Precision.HIGHEST) + b1_ref[...]
    mu = jnp.mean(h1, axis=0, keepdims=True)
    var = jnp.mean(jnp.square(h1 - mu), axis=0, keepdims=True)
    hn = (h1 - mu) * lax.rsqrt(var + 1e-5) * g_ref[...] + bt_ref[...]
    hn = jnp.maximum(hn, 0.0)
    h2 = jnp.dot(hn, w2_ref[...], preferred_element_type=jnp.float32,
                 precision=lax.Precision.HIGHEST) + b2_ref[...]
    o_ref[pl.ds(0, N), :] = jnp.maximum(h2, 0.0)
    o_ref[pl.ds(N, N2 - N), :] = jnp.zeros((N2 - N, H), jnp.float32)


_mlp = pl.pallas_call(
    _mlp_body,
    out_shape=jax.ShapeDtypeStruct((N2, H), jnp.float32),
)


def _pool_readout_body(h1_ref, h2_ref, h3_ref, b_ref, w1_ref, b1_ref,
                       w2_ref, b2_ref, o1_ref, o2_ref):
    gids = lax.broadcasted_iota(jnp.int32, (N, G), 1)
    onehot = (b_ref[...] == gids).astype(jnp.float32)
    dn = (((0,), (0,)), ((), ()))
    ones = jnp.ones((N, 1), jnp.float32)
    cnt = lax.dot_general(onehot, ones, dn,
                          preferred_element_type=jnp.float32,
                          precision=lax.Precision.HIGHEST)
    cnt = jnp.maximum(cnt, 1.0)
    s1 = lax.dot_general(onehot, h1_ref[pl.ds(0, N), :], dn,
                         preferred_element_type=jnp.float32,
                         precision=lax.Precision.HIGHEST)
    s2 = lax.dot_general(onehot, h2_ref[pl.ds(0, N), :], dn,
                         preferred_element_type=jnp.float32,
                         precision=lax.Precision.HIGHEST)
    s3 = lax.dot_general(onehot, h3_ref[pl.ds(0, N), :], dn,
                         preferred_element_type=jnp.float32,
                         precision=lax.Precision.HIGHEST)
    hcat = jnp.concatenate([s1, s2, s3], axis=1) / cnt
    hl = jnp.dot(hcat, w1_ref[...], preferred_element_type=jnp.float32,
                 precision=lax.Precision.HIGHEST) + b1_ref[...]
    hl = jnp.maximum(hl, 0.0)
    ho = jnp.dot(hl, w2_ref[...], preferred_element_type=jnp.float32,
                 precision=lax.Precision.HIGHEST) + b2_ref[...]
    o1_ref[...] = ho
    mx = jnp.max(ho, axis=1, keepdims=True)
    lse = jnp.log(jnp.sum(jnp.exp(ho - mx), axis=1, keepdims=True)) + mx
    o2_ref[...] = ho - lse


_pool_readout = pl.pallas_call(
    _pool_readout_body,
    out_shape=(jax.ShapeDtypeStruct((G, OUT), jnp.float32),
               jax.ShapeDtypeStruct((G, OUT), jnp.float32)),
)


def kernel(x, L0_W1, L0_b1, L0_g, L0_bt, L0_W2, L0_b2,
           L1_W1, L1_b1, L1_g, L1_bt, L1_W2, L1_b2,
           L2_W1, L2_b1, L2_g, L2_bt, L2_W2, L2_b2,
           lin1_W, lin1_b, lin2_W, lin2_b, edge_index, batch):
    pad = jnp.full((E_PAD - E,), N, jnp.int32)
    src3 = jnp.concatenate([edge_index[0], pad]).reshape(NW, NCH, CH)
    dst3 = jnp.concatenate([edge_index[1], pad]).reshape(NW, NCH, CH)
    params = [
        (L0_W1, L0_b1, L0_g, L0_bt, L0_W2, L0_b2),
        (L1_W1, L1_b1, L1_g, L1_bt, L1_W2, L1_b2),
        (L2_W1, L2_b1, L2_g, L2_bt, L2_W2, L2_b2),
    ]
    h = jnp.concatenate([x, jnp.zeros((N2 - N, D), jnp.float32)])
    hs = []
    for (W1, b1, g, bt, W2, b2) in params:
        agg = _sc_segsum(h, src3, dst3)
        h = _mlp(h, agg, W1, b1.reshape(1, H), g.reshape(1, H),
                 bt.reshape(1, H), W2, b2.reshape(1, H))
        hs.append(h)
    return _pool_readout(hs[0], hs[1], hs[2], batch.reshape(N, 1),
                         lin1_W, lin1_b.reshape(1, H * 3),
                         lin2_W, lin2_b.reshape(1, OUT))


# speedup vs baseline: 3.1368x; 2.9296x over previous
import functools

import jax
import jax.numpy as jnp
from jax import lax
from jax.experimental import pallas as pl
from jax.experimental.pallas import tpu as pltpu
from jax.experimental.pallas import tpu_sc as plsc

N = 10000
E = 320000
D = 128
H = 128
G = 64
OUT = 16

NC = 2
NS = 16
NW = NC * NS
CH = 80
NCH = 128
E_PAD = NW * NCH * CH
N2 = N + 16
N_PAD = 10240
ROWS_PT = N_PAD // NS
ZR = CH


def _sc_segsum_body(h_hbm, src_hbm, dst_hbm, out,
                    sidx, didx, sflat0, sflat1, rows0, rows1, acc,
                    sem0, sem1):
    cid = lax.axis_index("c")
    sid = lax.axis_index("s")
    wid = sid * NC + cid

    def zrow(i, carry):
        for c in range(H // 16):
            rows0[i, pl.ds(c * 16, 16)] = jnp.zeros((16,), jnp.float32)
        return carry
    lax.fori_loop(0, ZR, zrow, 0)
    for j in range(ROWS_PT // ZR):
        pltpu.sync_copy(rows0, acc.at[pl.ds(sid * ROWS_PT + j * ZR, ZR)])
    plsc.subcore_barrier()

    HN = NCH // 2
    for half in range(2):
        pltpu.sync_copy(src_hbm.at[wid, pl.ds(half * HN, HN)], sidx)
        pltpu.sync_copy(dst_hbm.at[wid, pl.ds(half * HN, HN)], didx)

        def stage(dst_flat, i):
            for c in range(CH // 16):
                dst_flat[pl.ds(c * 16, 16)] = sidx[i, pl.ds(c * 16, 16)]

        stage(sflat0, 0)
        pltpu.async_copy(h_hbm.at[sflat0], rows0, sem0)

        def body(b, carry):
            i0 = 2 * b
            stage(sflat1, i0 + 1)
            pltpu.async_copy(h_hbm.at[sflat1], rows1, sem1)
            pltpu.make_async_copy(h_hbm.at[sflat0], rows0, sem0).wait()
            pltpu.sync_copy(rows0, acc.at[didx.at[i0]], add=True)

            @pl.when(i0 + 2 < HN)
            def _():
                stage(sflat0, i0 + 2)
                pltpu.async_copy(h_hbm.at[sflat0], rows0, sem0)
            pltpu.make_async_copy(h_hbm.at[sflat1], rows1, sem1).wait()
            pltpu.sync_copy(rows1, acc.at[didx.at[i0 + 1]], add=True)
            return carry
        lax.fori_loop(0, HN // 2, body, 0)
    plsc.subcore_barrier()

    sl = pl.ds(sid * ROWS_PT, ROWS_PT)
    pltpu.sync_copy(acc.at[sl], out.at[cid, sl])


@functools.lru_cache(maxsize=None)
def _build_sc_segsum():
    mesh = plsc.VectorSubcoreMesh(core_axis_name="c", subcore_axis_name="s",
                                  num_cores=NC, num_subcores=NS)
    return pl.kernel(
        _sc_segsum_body,
        out_type=jax.ShapeDtypeStruct((2, N_PAD, H), jnp.float32),
        mesh=mesh,
        scratch_types=[
            pltpu.VMEM((NCH // 2, CH), jnp.int32),
            pltpu.VMEM((NCH // 2, CH), jnp.int32),
            pltpu.VMEM((CH,), jnp.int32),
            pltpu.VMEM((CH,), jnp.int32),
            pltpu.VMEM((CH, H), jnp.float32),
            pltpu.VMEM((CH, H), jnp.float32),
            pltpu.VMEM_SHARED((N_PAD, H), jnp.float32),
            pltpu.SemaphoreType.DMA,
            pltpu.SemaphoreType.DMA,
        ],
    )


def _sc_segsum(h, src3, dst3):
    return _build_sc_segsum()(h, src3, dst3)


def _mlp_body(h_ref, a_ref, w1_ref, b1_ref, g_ref, bt_ref,
              w2_ref, b2_ref, o_ref):
    hin = (h_ref[pl.ds(0, N), :] + a_ref[0, pl.ds(0, N), :]
           + a_ref[1, pl.ds(0, N), :])
    h1 = jnp.dot(hin, w1_ref[...], preferred_element_type=jnp.float32,
                 precision=lax.Precision.HIGHEST) + b1_ref[...]
    mu = jnp.mean(h1, axis=0, keepdims=True)
    var = jnp.mean(jnp.square(h1 - mu), axis=0, keepdims=True)
    hn = (h1 - mu) * lax.rsqrt(var + 1e-5) * g_ref[...] + bt_ref[...]
    hn = jnp.maximum(hn, 0.0)
    h2 = jnp.dot(hn, w2_ref[...], preferred_element_type=jnp.float32,
                 precision=lax.Precision.HIGHEST) + b2_ref[...]
    o_ref[pl.ds(0, N), :] = jnp.maximum(h2, 0.0)
    o_ref[pl.ds(N, N2 - N), :] = jnp.zeros((N2 - N, H), jnp.float32)


_mlp = pl.pallas_call(
    _mlp_body,
    out_shape=jax.ShapeDtypeStruct((N2, H), jnp.float32),
)


def _pool_readout_body(h1_ref, h2_ref, h3_ref, b_ref, w1_ref, b1_ref,
                       w2_ref, b2_ref, o1_ref, o2_ref):
    gids = lax.broadcasted_iota(jnp.int32, (N, G), 1)
    onehot = (b_ref[...] == gids).astype(jnp.float32)
    dn = (((0,), (0,)), ((), ()))
    ones = jnp.ones((N, 1), jnp.float32)
    cnt = lax.dot_general(onehot, ones, dn,
                          preferred_element_type=jnp.float32,
                          precision=lax.Precision.HIGHEST)
    cnt = jnp.maximum(cnt, 1.0)
    s1 = lax.dot_general(onehot, h1_ref[pl.ds(0, N), :], dn,
                         preferred_element_type=jnp.float32,
                         precision=lax.Precision.HIGHEST)
    s2 = lax.dot_general(onehot, h2_ref[pl.ds(0, N), :], dn,
                         preferred_element_type=jnp.float32,
                         precision=lax.Precision.HIGHEST)
    s3 = lax.dot_general(onehot, h3_ref[pl.ds(0, N), :], dn,
                         preferred_element_type=jnp.float32,
                         precision=lax.Precision.HIGHEST)
    hcat = jnp.concatenate([s1, s2, s3], axis=1) / cnt
    hl = jnp.dot(hcat, w1_ref[...], preferred_element_type=jnp.float32,
                 precision=lax.Precision.HIGHEST) + b1_ref[...]
    hl = jnp.maximum(hl, 0.0)
    ho = jnp.dot(hl, w2_ref[...], preferred_element_type=jnp.float32,
                 precision=lax.Precision.HIGHEST) + b2_ref[...]
    o1_ref[...] = ho
    mx = jnp.max(ho, axis=1, keepdims=True)
    lse = jnp.log(jnp.sum(jnp.exp(ho - mx), axis=1, keepdims=True)) + mx
    o2_ref[...] = ho - lse


_pool_readout = pl.pallas_call(
    _pool_readout_body,
    out_shape=(jax.ShapeDtypeStruct((G, OUT), jnp.float32),
               jax.ShapeDtypeStruct((G, OUT), jnp.float32)),
)


def kernel(x, L0_W1, L0_b1, L0_g, L0_bt, L0_W2, L0_b2,
           L1_W1, L1_b1, L1_g, L1_bt, L1_W2, L1_b2,
           L2_W1, L2_b1, L2_g, L2_bt, L2_W2, L2_b2,
           lin1_W, lin1_b, lin2_W, lin2_b, edge_index, batch):
    ppt = (E_PAD - E) // NW
    pad_s = jnp.broadcast_to(
        N + (jnp.arange(ppt, dtype=jnp.int32) % (N2 - N)), (NW, ppt))
    pad_d = jnp.broadcast_to(
        N + jnp.arange(ppt, dtype=jnp.int32), (NW, ppt))
    src3 = jnp.concatenate(
        [edge_index[0].reshape(NW, E // NW), pad_s], axis=1
    ).reshape(NW, NCH, CH)
    dst3 = jnp.concatenate(
        [edge_index[1].reshape(NW, E // NW), pad_d], axis=1
    ).reshape(NW, NCH, CH)
    params = [
        (L0_W1, L0_b1, L0_g, L0_bt, L0_W2, L0_b2),
        (L1_W1, L1_b1, L1_g, L1_bt, L1_W2, L1_b2),
        (L2_W1, L2_b1, L2_g, L2_bt, L2_W2, L2_b2),
    ]
    h = jnp.concatenate([x, jnp.zeros((N2 - N, D), jnp.float32)])
    hs = []
    for (W1, b1, g, bt, W2, b2) in params:
        agg = _sc_segsum(h, src3, dst3)
        h = _mlp(h, agg, W1, b1.reshape(1, H), g.reshape(1, H),
                 bt.reshape(1, H), W2, b2.reshape(1, H))
        hs.append(h)
    return _pool_readout(hs[0], hs[1], hs[2], batch.reshape(N, 1),
                         lin1_W, lin1_b.reshape(1, H * 3),
                         lin2_W, lin2_b.reshape(1, OUT))


# speedup vs baseline: 3.4288x; 1.0931x over previous
import functools

import jax
import jax.numpy as jnp
from jax import lax
from jax.experimental import pallas as pl
from jax.experimental.pallas import tpu as pltpu
from jax.experimental.pallas import tpu_sc as plsc

N = 10000
E = 320000
D = 128
H = 128
G = 64
OUT = 16

NC = 2
NS = 16
NW = NC * NS
CH = 80
NCH = 128
E_PAD = NW * NCH * CH
N2 = N + 16
N_PAD = 10240
ROWS_PT = N_PAD // NS
ZR = CH


def _sc_segsum_body(h_hbm, src_hbm, dst_hbm, out,
                    sidx, didx, rows0, rows1, rows2, rows3, acc,
                    sem0, sem1, sem2, sem3):
    cid = lax.axis_index("c")
    sid = lax.axis_index("s")
    wid = sid * NC + cid

    def zrow(i, carry):
        for c in range(H // 16):
            rows0[i, pl.ds(c * 16, 16)] = jnp.zeros((16,), jnp.float32)
        return carry
    lax.fori_loop(0, ZR, zrow, 0)
    for j in range(ROWS_PT // ZR):
        pltpu.sync_copy(rows0, acc.at[pl.ds(sid * ROWS_PT + j * ZR, ZR)])
    plsc.subcore_barrier()

    rows = (rows0, rows1, rows2, rows3)
    sems = (sem0, sem1, sem2, sem3)
    QN = NCH // 4
    for q in range(4):
        pltpu.sync_copy(src_hbm.at[wid, pl.ds(q * QN, QN)], sidx)
        pltpu.sync_copy(dst_hbm.at[wid, pl.ds(q * QN, QN)], didx)
        for j in range(3):
            pltpu.async_copy(h_hbm.at[sidx.at[j]], rows[j], sems[j])

        def body(b, carry):
            i0 = 4 * b
            for j in range(4):
                i = i0 + j
                pltpu.make_async_copy(h_hbm.at[sidx.at[i]], rows[j],
                                      sems[j]).wait()
                pltpu.sync_copy(rows[j], acc.at[didx.at[i]], add=True)

                def fire(i=i, j=j):
                    pltpu.async_copy(h_hbm.at[sidx.at[i + 3]],
                                     rows[(j + 3) % 4], sems[(j + 3) % 4])
                pl.when(i + 3 < QN)(fire)
            return carry
        lax.fori_loop(0, QN // 4, body, 0)
    plsc.subcore_barrier()

    sl = pl.ds(sid * ROWS_PT, ROWS_PT)
    pltpu.sync_copy(acc.at[sl], out.at[cid, sl])


@functools.lru_cache(maxsize=None)
def _build_sc_segsum():
    mesh = plsc.VectorSubcoreMesh(core_axis_name="c", subcore_axis_name="s",
                                  num_cores=NC, num_subcores=NS)
    return pl.kernel(
        _sc_segsum_body,
        out_type=jax.ShapeDtypeStruct((2, N_PAD, H), jnp.float32),
        mesh=mesh,
        scratch_types=[
            pltpu.VMEM((NCH // 4, CH), jnp.int32),
            pltpu.VMEM((NCH // 4, CH), jnp.int32),
            pltpu.VMEM((CH, H), jnp.float32),
            pltpu.VMEM((CH, H), jnp.float32),
            pltpu.VMEM((CH, H), jnp.float32),
            pltpu.VMEM((CH, H), jnp.float32),
            pltpu.VMEM_SHARED((N_PAD, H), jnp.float32),
            pltpu.SemaphoreType.DMA,
            pltpu.SemaphoreType.DMA,
            pltpu.SemaphoreType.DMA,
            pltpu.SemaphoreType.DMA,
        ],
    )


def _sc_segsum(h, src3, dst3):
    return _build_sc_segsum()(h, src3, dst3)


def _mlp_body(h_ref, a_ref, w1_ref, b1_ref, g_ref, bt_ref,
              w2_ref, b2_ref, o_ref):
    hin = (h_ref[pl.ds(0, N), :] + a_ref[0, pl.ds(0, N), :]
           + a_ref[1, pl.ds(0, N), :])
    h1 = jnp.dot(hin, w1_ref[...], preferred_element_type=jnp.float32,
                 precision=lax.Precision.HIGHEST) + b1_ref[...]
    mu = jnp.mean(h1, axis=0, keepdims=True)
    var = jnp.mean(jnp.square(h1 - mu), axis=0, keepdims=True)
    hn = (h1 - mu) * lax.rsqrt(var + 1e-5) * g_ref[...] + bt_ref[...]
    hn = jnp.maximum(hn, 0.0)
    h2 = jnp.dot(hn, w2_ref[...], preferred_element_type=jnp.float32,
                 precision=lax.Precision.HIGHEST) + b2_ref[...]
    o_ref[pl.ds(0, N), :] = jnp.maximum(h2, 0.0)
    o_ref[pl.ds(N, N2 - N), :] = jnp.zeros((N2 - N, H), jnp.float32)


_mlp = pl.pallas_call(
    _mlp_body,
    out_shape=jax.ShapeDtypeStruct((N2, H), jnp.float32),
)


def _pool_readout_body(h1_ref, h2_ref, h3_ref, b_ref, w1_ref, b1_ref,
                       w2_ref, b2_ref, o1_ref, o2_ref):
    gids = lax.broadcasted_iota(jnp.int32, (N, G), 1)
    onehot = (b_ref[...] == gids).astype(jnp.float32)
    dn = (((0,), (0,)), ((), ()))
    ones = jnp.ones((N, 1), jnp.float32)
    cnt = lax.dot_general(onehot, ones, dn,
                          preferred_element_type=jnp.float32,
                          precision=lax.Precision.HIGHEST)
    cnt = jnp.maximum(cnt, 1.0)
    s1 = lax.dot_general(onehot, h1_ref[pl.ds(0, N), :], dn,
                         preferred_element_type=jnp.float32,
                         precision=lax.Precision.HIGHEST)
    s2 = lax.dot_general(onehot, h2_ref[pl.ds(0, N), :], dn,
                         preferred_element_type=jnp.float32,
                         precision=lax.Precision.HIGHEST)
    s3 = lax.dot_general(onehot, h3_ref[pl.ds(0, N), :], dn,
                         preferred_element_type=jnp.float32,
                         precision=lax.Precision.HIGHEST)
    hcat = jnp.concatenate([s1, s2, s3], axis=1) / cnt
    hl = jnp.dot(hcat, w1_ref[...], preferred_element_type=jnp.float32,
                 precision=lax.Precision.HIGHEST) + b1_ref[...]
    hl = jnp.maximum(hl, 0.0)
    ho = jnp.dot(hl, w2_ref[...], preferred_element_type=jnp.float32,
                 precision=lax.Precision.HIGHEST) + b2_ref[...]
    o1_ref[...] = ho
    mx = jnp.max(ho, axis=1, keepdims=True)
    lse = jnp.log(jnp.sum(jnp.exp(ho - mx), axis=1, keepdims=True)) + mx
    o2_ref[...] = ho - lse


_pool_readout = pl.pallas_call(
    _pool_readout_body,
    out_shape=(jax.ShapeDtypeStruct((G, OUT), jnp.float32),
               jax.ShapeDtypeStruct((G, OUT), jnp.float32)),
)


def kernel(x, L0_W1, L0_b1, L0_g, L0_bt, L0_W2, L0_b2,
           L1_W1, L1_b1, L1_g, L1_bt, L1_W2, L1_b2,
           L2_W1, L2_b1, L2_g, L2_bt, L2_W2, L2_b2,
           lin1_W, lin1_b, lin2_W, lin2_b, edge_index, batch):
    ppt = (E_PAD - E) // NW
    pad_s = jnp.broadcast_to(
        N + (jnp.arange(ppt, dtype=jnp.int32) % (N2 - N)), (NW, ppt))
    pad_d = jnp.broadcast_to(
        N + jnp.arange(ppt, dtype=jnp.int32), (NW, ppt))
    src3 = jnp.concatenate(
        [edge_index[0].reshape(NW, E // NW), pad_s], axis=1
    ).reshape(NW, NCH, CH)
    dst3 = jnp.concatenate(
        [edge_index[1].reshape(NW, E // NW), pad_d], axis=1
    ).reshape(NW, NCH, CH)
    params = [
        (L0_W1, L0_b1, L0_g, L0_bt, L0_W2, L0_b2),
        (L1_W1, L1_b1, L1_g, L1_bt, L1_W2, L1_b2),
        (L2_W1, L2_b1, L2_g, L2_bt, L2_W2, L2_b2),
    ]
    h = jnp.concatenate([x, jnp.zeros((N2 - N, D), jnp.float32)])
    hs = []
    for (W1, b1, g, bt, W2, b2) in params:
        agg = _sc_segsum(h, src3, dst3)
        h = _mlp(h, agg, W1, b1.reshape(1, H), g.reshape(1, H),
                 bt.reshape(1, H), W2, b2.reshape(1, H))
        hs.append(h)
    return _pool_readout(hs[0], hs[1], hs[2], batch.reshape(N, 1),
                         lin1_W, lin1_b.reshape(1, H * 3),
                         lin2_W, lin2_b.reshape(1, OUT))


# speedup vs baseline: 3.4910x; 1.0181x over previous
import functools

import jax
import jax.numpy as jnp
from jax import lax
from jax.experimental import pallas as pl
from jax.experimental.pallas import tpu as pltpu
from jax.experimental.pallas import tpu_sc as plsc

N = 10000
E = 320000
D = 128
H = 128
G = 64
OUT = 16

NC = 2
NS = 16
NW = NC * NS
CH = 80
NCH = 128
E_PAD = NW * NCH * CH
N2 = N + 16
N_PAD = 10240
ROWS_PT = N_PAD // NS
ZR = CH


def _sc_segsum_body(h_hbm, src_hbm, dst_hbm, out,
                    sidx, didx, rows0, rows1, rows2, rows3, acc,
                    sem0, sem1, sem2, sem3):
    cid = lax.axis_index("c")
    sid = lax.axis_index("s")
    wid = sid * NC + cid

    def zrow(i, carry):
        for c in range(H // 16):
            rows0[i, pl.ds(c * 16, 16)] = jnp.zeros((16,), jnp.float32)
        return carry
    lax.fori_loop(0, ZR, zrow, 0)
    for j in range(ROWS_PT // ZR):
        pltpu.sync_copy(rows0, acc.at[pl.ds(sid * ROWS_PT + j * ZR, ZR)])
    plsc.subcore_barrier()

    rows = (rows0, rows1, rows2, rows3)
    sems = (sem0, sem1, sem2, sem3)
    QN = NCH // 4
    for q in range(4):
        pltpu.sync_copy(src_hbm.at[wid, pl.ds(q * QN, QN)], sidx)
        pltpu.sync_copy(dst_hbm.at[wid, pl.ds(q * QN, QN)], didx)
        for j in range(3):
            pltpu.async_copy(h_hbm.at[sidx.at[j]], rows[j], sems[j])

        def body(b, carry):
            i0 = 4 * b
            for j in range(4):
                i = i0 + j
                pltpu.make_async_copy(h_hbm.at[sidx.at[i]], rows[j],
                                      sems[j]).wait()
                pltpu.sync_copy(rows[j], acc.at[didx.at[i]], add=True)

                def fire(i=i, j=j):
                    pltpu.async_copy(h_hbm.at[sidx.at[i + 3]],
                                     rows[(j + 3) % 4], sems[(j + 3) % 4])
                pl.when(i + 3 < QN)(fire)
            return carry
        lax.fori_loop(0, QN // 4, body, 0)
    plsc.subcore_barrier()

    sl = pl.ds(sid * ROWS_PT, ROWS_PT)
    pltpu.sync_copy(acc.at[sl], out.at[cid, sl])


@functools.lru_cache(maxsize=None)
def _build_sc_segsum():
    mesh = plsc.VectorSubcoreMesh(core_axis_name="c", subcore_axis_name="s",
                                  num_cores=NC, num_subcores=NS)
    return pl.kernel(
        _sc_segsum_body,
        out_type=jax.ShapeDtypeStruct((2, N_PAD, H), jnp.float32),
        mesh=mesh,
        scratch_types=[
            pltpu.VMEM((NCH // 4, CH), jnp.int32),
            pltpu.VMEM((NCH // 4, CH), jnp.int32),
            pltpu.VMEM((CH, H), jnp.float32),
            pltpu.VMEM((CH, H), jnp.float32),
            pltpu.VMEM((CH, H), jnp.float32),
            pltpu.VMEM((CH, H), jnp.float32),
            pltpu.VMEM_SHARED((N_PAD, H), jnp.float32),
            pltpu.SemaphoreType.DMA,
            pltpu.SemaphoreType.DMA,
            pltpu.SemaphoreType.DMA,
            pltpu.SemaphoreType.DMA,
        ],
    )


def _sc_segsum(h, src3, dst3):
    return _build_sc_segsum()(h, src3, dst3)


def _mlp_body(h_ref, a_ref, b_ref, w1_ref, b1_ref, g_ref, bt_ref,
              w2_ref, b2_ref, o_ref, s_ref):
    hin = (h_ref[pl.ds(0, N), :] + a_ref[0, pl.ds(0, N), :]
           + a_ref[1, pl.ds(0, N), :])
    h1 = jnp.dot(hin, w1_ref[...], preferred_element_type=jnp.float32,
                 precision=lax.Precision.HIGHEST) + b1_ref[...]
    mu = jnp.mean(h1, axis=0, keepdims=True)
    var = jnp.mean(jnp.square(h1 - mu), axis=0, keepdims=True)
    hn = (h1 - mu) * lax.rsqrt(var + 1e-5) * g_ref[...] + bt_ref[...]
    hn = jnp.maximum(hn, 0.0)
    h2 = jnp.dot(hn, w2_ref[...], preferred_element_type=jnp.float32,
                 precision=lax.Precision.HIGHEST) + b2_ref[...]
    hout = jnp.maximum(h2, 0.0)
    o_ref[pl.ds(0, N), :] = hout
    o_ref[pl.ds(N, N2 - N), :] = jnp.zeros((N2 - N, H), jnp.float32)
    gids = lax.broadcasted_iota(jnp.int32, (N, G), 1)
    onehot = (b_ref[...] == gids).astype(jnp.float32)
    dn = (((0,), (0,)), ((), ()))
    s_ref[...] = lax.dot_general(onehot, hout, dn,
                                 preferred_element_type=jnp.float32,
                                 precision=lax.Precision.HIGHEST)


_mlp = pl.pallas_call(
    _mlp_body,
    out_shape=(jax.ShapeDtypeStruct((N2, H), jnp.float32),
               jax.ShapeDtypeStruct((G, H), jnp.float32)),
)


def _readout_body(s1_ref, s2_ref, s3_ref, b_ref, w1_ref, b1_ref,
                  w2_ref, b2_ref, o1_ref, o2_ref):
    gids = lax.broadcasted_iota(jnp.int32, (N, G), 1)
    onehot = (b_ref[...] == gids).astype(jnp.float32)
    dn = (((0,), (0,)), ((), ()))
    ones = jnp.ones((N, 1), jnp.float32)
    cnt = lax.dot_general(onehot, ones, dn,
                          preferred_element_type=jnp.float32,
                          precision=lax.Precision.HIGHEST)
    cnt = jnp.maximum(cnt, 1.0)
    hcat = jnp.concatenate([s1_ref[...], s2_ref[...], s3_ref[...]],
                           axis=1) / cnt
    hl = jnp.dot(hcat, w1_ref[...], preferred_element_type=jnp.float32,
                 precision=lax.Precision.HIGHEST) + b1_ref[...]
    hl = jnp.maximum(hl, 0.0)
    ho = jnp.dot(hl, w2_ref[...], preferred_element_type=jnp.float32,
                 precision=lax.Precision.HIGHEST) + b2_ref[...]
    o1_ref[...] = ho
    mx = jnp.max(ho, axis=1, keepdims=True)
    lse = jnp.log(jnp.sum(jnp.exp(ho - mx), axis=1, keepdims=True)) + mx
    o2_ref[...] = ho - lse


_readout = pl.pallas_call(
    _readout_body,
    out_shape=(jax.ShapeDtypeStruct((G, OUT), jnp.float32),
               jax.ShapeDtypeStruct((G, OUT), jnp.float32)),
)


def kernel(x, L0_W1, L0_b1, L0_g, L0_bt, L0_W2, L0_b2,
           L1_W1, L1_b1, L1_g, L1_bt, L1_W2, L1_b2,
           L2_W1, L2_b1, L2_g, L2_bt, L2_W2, L2_b2,
           lin1_W, lin1_b, lin2_W, lin2_b, edge_index, batch):
    ppt = (E_PAD - E) // NW
    pad_s = jnp.broadcast_to(
        N + (jnp.arange(ppt, dtype=jnp.int32) % (N2 - N)), (NW, ppt))
    pad_d = jnp.broadcast_to(
        N + jnp.arange(ppt, dtype=jnp.int32), (NW, ppt))
    src3 = jnp.concatenate(
        [edge_index[0].reshape(NW, E // NW), pad_s], axis=1
    ).reshape(NW, NCH, CH)
    dst3 = jnp.concatenate(
        [edge_index[1].reshape(NW, E // NW), pad_d], axis=1
    ).reshape(NW, NCH, CH)
    params = [
        (L0_W1, L0_b1, L0_g, L0_bt, L0_W2, L0_b2),
        (L1_W1, L1_b1, L1_g, L1_bt, L1_W2, L1_b2),
        (L2_W1, L2_b1, L2_g, L2_bt, L2_W2, L2_b2),
    ]
    h = jnp.concatenate([x, jnp.zeros((N2 - N, D), jnp.float32)])
    b2d = batch.reshape(N, 1)
    ss = []
    for (W1, b1, g, bt, W2, b2) in params:
        agg = _sc_segsum(h, src3, dst3)
        h, s = _mlp(h, agg, b2d, W1, b1.reshape(1, H), g.reshape(1, H),
                    bt.reshape(1, H), W2, b2.reshape(1, H))
        ss.append(s)
    return _readout(ss[0], ss[1], ss[2], b2d,
                    lin1_W, lin1_b.reshape(1, H * 3),
                    lin2_W, lin2_b.reshape(1, OUT))
